# baseline (speedup 1.0000x reference)
import functools

import jax
import jax.numpy as jnp
from jax import lax
from jax.experimental import pallas as pl
from jax.experimental.pallas import tpu as pltpu
from jax.experimental.pallas import tpu_sc as plsc

DIM = 256
K = 8192
N_TOKENS = 8192
TOK_BLK = 512
N_BLK = N_TOKENS // TOK_BLK
K_OUT_BLK = K // N_BLK
GRP = 128
CHUNK = 128


def _fused_body(cb_ref, w_ref, x_ref, colf_ref,
                imp_hbm_ref, idx_ref, loss_ref,
                imp_ref, c2_ref):
    i = pl.program_id(0)

    @pl.when(i == 0)
    def _init():
        imp = lax.dot_general(cb_ref[...], w_ref[...],
                              (((1,), (1,)), ((), ())),
                              preferred_element_type=jnp.float32)
        imp_ref[...] = imp
        c2_ref[...] = 0.5 * jnp.sum(imp * imp, axis=1, keepdims=True
                                    ).reshape(1, K)
        loss_ref[0, 0] = 0.0

    imp_hbm_ref[...] = imp_ref[pl.ds(i * K_OUT_BLK, K_OUT_BLK), :]

    x = x_ref[...]
    x2h = 0.5 * jnp.sum(x * x, axis=1, keepdims=True)
    imp = imp_ref[...]
    c2h = c2_ref[...]
    lanef = colf_ref[0:1, 0:CHUNK]

    for g in range(TOK_BLK // GRP):
        xg = x[g * GRP:(g + 1) * GRP, :]
        x2g = x2h[g * GRP:(g + 1) * GRP, :]
        cross = lax.dot_general(xg, imp,
                                (((1,), (1,)), ((), ())),
                                preferred_element_type=jnp.float32)
        runval = (x2g - cross[:, 0:CHUNK]) + c2h[0:1, 0:CHUNK]
        runc = jnp.zeros((GRP, CHUNK), jnp.float32)
        for c in range(1, K // CHUNK):
            d2c = (x2g - cross[:, c * CHUNK:(c + 1) * CHUNK]) \
                + c2h[0:1, c * CHUNK:(c + 1) * CHUNK]
            take = d2c < runval
            runval = jnp.where(take, d2c, runval)
            runc = jnp.where(take, jnp.float32(c), runc)
        mg = jnp.min(runval, axis=1, keepdims=True)
        idxfull = runc * jnp.float32(CHUNK) + lanef
        idxf = jnp.min(jnp.where(runval == mg, idxfull, jnp.float32(K * 2)),
                       axis=1)
        idx_ref[0, 0, g * GRP:(g + 1) * GRP] = idxf.astype(jnp.int32)
        loss_ref[0, 0] += jnp.sum(mg)

    @pl.when(i == N_BLK - 1)
    def _scale():
        loss_ref[0, 0] = loss_ref[0, 0] * jnp.float32(2.0 / N_TOKENS)


def _fused_call(xf, codebook, W):
    colf = jnp.arange(K, dtype=jnp.float32).reshape(1, K)
    return pl.pallas_call(
        _fused_body,
        grid=(N_BLK,),
        in_specs=[
            pl.BlockSpec((K, DIM), lambda i: (0, 0)),
            pl.BlockSpec((DIM, DIM), lambda i: (0, 0)),
            pl.BlockSpec((TOK_BLK, DIM), lambda i: (i, 0)),
            pl.BlockSpec((1, K), lambda i: (0, 0)),
        ],
        out_specs=[
            pl.BlockSpec((K_OUT_BLK, DIM), lambda i: (i, 0)),
            pl.BlockSpec((1, 1, TOK_BLK), lambda i: (i, 0, 0)),
            pl.BlockSpec((1, 1), lambda i: (0, 0), memory_space=pltpu.SMEM),
        ],
        out_shape=[
            jax.ShapeDtypeStruct((K, DIM), jnp.float32),
            jax.ShapeDtypeStruct((N_BLK, 1, TOK_BLK), jnp.int32),
            jax.ShapeDtypeStruct((1, 1), jnp.float32),
        ],
        scratch_shapes=[
            pltpu.VMEM((K, DIM), jnp.float32),
            pltpu.VMEM((1, K), jnp.float32),
        ],
    )(codebook, W, xf, colf)


def _make_gather():
    info = plsc.get_sparse_core_info()
    nc, ns = info.num_cores, info.num_subcores
    nw = nc * ns
    b_per_w = N_TOKENS // nw
    mesh = plsc.VectorSubcoreMesh(core_axis_name="c", subcore_axis_name="s")

    @functools.partial(
        pl.kernel,
        mesh=mesh,
        out_type=jax.ShapeDtypeStruct((N_TOKENS, DIM), jnp.float32),
        scratch_types=[
            pltpu.VMEM((b_per_w,), jnp.int32),
            pltpu.VMEM((b_per_w, DIM), jnp.float32),
            pltpu.SemaphoreType.DMA,
        ],
    )
    def gather(table_hbm, idx_hbm, out_hbm, idx_v, rows_v, sem):
        wid = lax.axis_index("s") * nc + lax.axis_index("c")
        base = wid * b_per_w
        pltpu.sync_copy(idx_hbm.at[pl.ds(base, b_per_w)], idx_v)
        pltpu.async_copy(table_hbm.at[idx_v], rows_v, sem).wait()
        pltpu.sync_copy(rows_v, out_hbm.at[pl.ds(base, b_per_w)])

    return gather


def kernel(x, codebook, W):
    b, n, d = x.shape
    xf = x.reshape(b * n, d)
    implicit, idx3, loss = _fused_call(xf, codebook, W)
    idx = idx3.reshape(N_TOKENS)
    q = _make_gather()(implicit, idx)
    return q.reshape(x.shape), idx.reshape(b, n), loss[0, 0]


# speedup vs baseline: 1.2750x; 1.2750x over previous
import functools

import jax
import jax.numpy as jnp
from jax import lax
from jax.experimental import pallas as pl
from jax.experimental.pallas import tpu as pltpu
from jax.experimental.pallas import tpu_sc as plsc

DIM = 256
K = 8192
N_TOKENS = 8192
TOK_BLK = 512
N_BLK = N_TOKENS // TOK_BLK
K_OUT_BLK = K // N_BLK
GRP = 64
CHUNK = 128


def _fused_body(cb_ref, w_ref, x_ref, colf_ref,
                imp_hbm_ref, idx_ref, loss_ref,
                imp_ref, c2_ref):
    i = pl.program_id(0)

    @pl.when(i == 0)
    def _init():
        imp = lax.dot_general(cb_ref[...], w_ref[...],
                              (((1,), (1,)), ((), ())),
                              preferred_element_type=jnp.float32)
        imp_ref[...] = imp
        c2_ref[...] = 0.5 * jnp.sum(imp * imp, axis=1, keepdims=True
                                    ).reshape(1, K)
        loss_ref[0, 0] = 0.0

    imp_hbm_ref[...] = imp_ref[pl.ds(i * K_OUT_BLK, K_OUT_BLK), :]

    x = x_ref[...]
    x2h = 0.5 * jnp.sum(x * x, axis=1, keepdims=True)
    imp = imp_ref[...]
    c2h = c2_ref[...]
    lanef = colf_ref[0:1, 0:CHUNK]

    cross_full = lax.dot_general(x, imp,
                                 (((1,), (1,)), ((), ())),
                                 preferred_element_type=jnp.float32)
    for g in range(TOK_BLK // GRP):
        cross = cross_full[g * GRP:(g + 1) * GRP, :]
        x2g = x2h[g * GRP:(g + 1) * GRP, :]
        runval = (x2g - cross[:, 0:CHUNK]) + c2h[0:1, 0:CHUNK]
        runc = jnp.zeros((GRP, CHUNK), jnp.float32)
        for c in range(1, K // CHUNK):
            d2c = (x2g - cross[:, c * CHUNK:(c + 1) * CHUNK]) \
                + c2h[0:1, c * CHUNK:(c + 1) * CHUNK]
            take = d2c < runval
            runval = jnp.where(take, d2c, runval)
            runc = jnp.where(take, jnp.float32(c), runc)
        mg = jnp.min(runval, axis=1, keepdims=True)
        idxfull = runc * jnp.float32(CHUNK) + lanef
        idxf = jnp.min(jnp.where(runval == mg, idxfull, jnp.float32(K * 2)),
                       axis=1)
        idx_ref[0, 0, g * GRP:(g + 1) * GRP] = idxf.astype(jnp.int32)
        loss_ref[0, 0] += jnp.sum(mg)

    @pl.when(i == N_BLK - 1)
    def _scale():
        loss_ref[0, 0] = loss_ref[0, 0] * jnp.float32(2.0 / N_TOKENS)


def _fused_call(xf, codebook, W):
    colf = jnp.arange(K, dtype=jnp.float32).reshape(1, K)
    return pl.pallas_call(
        _fused_body,
        grid=(N_BLK,),
        in_specs=[
            pl.BlockSpec((K, DIM), lambda i: (0, 0)),
            pl.BlockSpec((DIM, DIM), lambda i: (0, 0)),
            pl.BlockSpec((TOK_BLK, DIM), lambda i: (i, 0)),
            pl.BlockSpec((1, K), lambda i: (0, 0)),
        ],
        out_specs=[
            pl.BlockSpec((K_OUT_BLK, DIM), lambda i: (i, 0)),
            pl.BlockSpec((1, 1, TOK_BLK), lambda i: (i, 0, 0)),
            pl.BlockSpec((1, 1), lambda i: (0, 0), memory_space=pltpu.SMEM),
        ],
        out_shape=[
            jax.ShapeDtypeStruct((K, DIM), jnp.float32),
            jax.ShapeDtypeStruct((N_BLK, 1, TOK_BLK), jnp.int32),
            jax.ShapeDtypeStruct((1, 1), jnp.float32),
        ],
        scratch_shapes=[
            pltpu.VMEM((K, DIM), jnp.float32),
            pltpu.VMEM((1, K), jnp.float32),
        ],
    )(codebook, W, xf, colf)


def _make_gather():
    info = plsc.get_sparse_core_info()
    nc, ns = info.num_cores, info.num_subcores
    nw = nc * ns
    b_per_w = N_TOKENS // nw
    mesh = plsc.VectorSubcoreMesh(core_axis_name="c", subcore_axis_name="s")

    @functools.partial(
        pl.kernel,
        mesh=mesh,
        out_type=jax.ShapeDtypeStruct((N_TOKENS, DIM), jnp.float32),
        scratch_types=[
            pltpu.VMEM((b_per_w,), jnp.int32),
            pltpu.VMEM((b_per_w, DIM), jnp.float32),
            pltpu.SemaphoreType.DMA,
        ],
    )
    def gather(table_hbm, idx_hbm, out_hbm, idx_v, rows_v, sem):
        wid = lax.axis_index("s") * nc + lax.axis_index("c")
        base = wid * b_per_w
        pltpu.sync_copy(idx_hbm.at[pl.ds(base, b_per_w)], idx_v)
        pltpu.async_copy(table_hbm.at[idx_v], rows_v, sem).wait()
        pltpu.sync_copy(rows_v, out_hbm.at[pl.ds(base, b_per_w)])

    return gather


def kernel(x, codebook, W):
    b, n, d = x.shape
    xf = x.reshape(b * n, d)
    implicit, idx3, loss = _fused_call(xf, codebook, W)
    idx = idx3.reshape(N_TOKENS)
    q = _make_gather()(implicit, idx)
    return q.reshape(x.shape), idx.reshape(b, n), loss[0, 0]


# speedup vs baseline: 1.3227x; 1.0374x over previous
import functools

import jax
import jax.numpy as jnp
from jax import lax
from jax.experimental import pallas as pl
from jax.experimental.pallas import tpu as pltpu
from jax.experimental.pallas import tpu_sc as plsc

DIM = 256
K = 8192
N_TOKENS = 8192
TOK_BLK = 512
N_BLK = N_TOKENS // TOK_BLK
K_OUT_BLK = K // N_BLK
GRP = 64
CHUNK = 128


def _fused_body(cb_ref, w_ref, x_ref, colf_ref,
                imp_hbm_ref, idx_ref, loss_ref,
                imp_ref, c2_ref):
    i = pl.program_id(0)

    @pl.when(i == 0)
    def _init():
        imp = lax.dot_general(cb_ref[...], w_ref[...],
                              (((1,), (1,)), ((), ())),
                              preferred_element_type=jnp.float32)
        imp_ref[...] = imp
        imp2 = imp * imp
        c2_ref[...] = 0.5 * jnp.sum(imp2.reshape(K // CHUNK, CHUNK, DIM),
                                    axis=2)
        loss_ref[0, 0] = 0.0

    imp_hbm_ref[...] = imp_ref[pl.ds(i * K_OUT_BLK, K_OUT_BLK), :]

    x = x_ref[...]
    x2h = 0.5 * jnp.sum(x * x, axis=1, keepdims=True)
    imp = imp_ref[...]
    lanef = colf_ref[...]

    cross_full = lax.dot_general(x, imp,
                                 (((1,), (1,)), ((), ())),
                                 preferred_element_type=jnp.float32)
    for g in range(TOK_BLK // GRP):
        cross = cross_full[g * GRP:(g + 1) * GRP, :]
        x2g = x2h[g * GRP:(g + 1) * GRP, :] + jnp.zeros((GRP, CHUNK),
                                                        jnp.float32)
        runval = (x2g - cross[:, 0:CHUNK]) + c2_ref[0:1, :]
        runc = jnp.zeros((GRP, CHUNK), jnp.float32)
        for c in range(1, K // CHUNK):
            d2c = (x2g - cross[:, c * CHUNK:(c + 1) * CHUNK]) \
                + c2_ref[c:c + 1, :]
            take = d2c < runval
            runval = jnp.where(take, d2c, runval)
            runc = jnp.where(take, jnp.float32(c), runc)
        mg = jnp.min(runval, axis=1, keepdims=True)
        idxfull = runc * jnp.float32(CHUNK) + lanef
        idxf = jnp.min(jnp.where(runval == mg, idxfull, jnp.float32(K * 2)),
                       axis=1)
        idx_ref[0, 0, g * GRP:(g + 1) * GRP] = idxf.astype(jnp.int32)
        loss_ref[0, 0] += jnp.sum(mg)

    @pl.when(i == N_BLK - 1)
    def _scale():
        loss_ref[0, 0] = loss_ref[0, 0] * jnp.float32(2.0 / N_TOKENS)


def _fused_call(xf, codebook, W):
    colf = jnp.arange(CHUNK, dtype=jnp.float32).reshape(1, CHUNK)
    return pl.pallas_call(
        _fused_body,
        grid=(N_BLK,),
        in_specs=[
            pl.BlockSpec((K, DIM), lambda i: (0, 0)),
            pl.BlockSpec((DIM, DIM), lambda i: (0, 0)),
            pl.BlockSpec((TOK_BLK, DIM), lambda i: (i, 0)),
            pl.BlockSpec((1, CHUNK), lambda i: (0, 0)),
        ],
        out_specs=[
            pl.BlockSpec((K_OUT_BLK, DIM), lambda i: (i, 0)),
            pl.BlockSpec((1, 1, TOK_BLK), lambda i: (i, 0, 0)),
            pl.BlockSpec((1, 1), lambda i: (0, 0), memory_space=pltpu.SMEM),
        ],
        out_shape=[
            jax.ShapeDtypeStruct((K, DIM), jnp.float32),
            jax.ShapeDtypeStruct((N_BLK, 1, TOK_BLK), jnp.int32),
            jax.ShapeDtypeStruct((1, 1), jnp.float32),
        ],
        scratch_shapes=[
            pltpu.VMEM((K, DIM), jnp.float32),
            pltpu.VMEM((K // CHUNK, CHUNK), jnp.float32),
        ],
    )(codebook, W, xf, colf)


def _make_gather():
    info = plsc.get_sparse_core_info()
    nc, ns = info.num_cores, info.num_subcores
    nw = nc * ns
    b_per_w = N_TOKENS // nw
    mesh = plsc.VectorSubcoreMesh(core_axis_name="c", subcore_axis_name="s")

    @functools.partial(
        pl.kernel,
        mesh=mesh,
        out_type=jax.ShapeDtypeStruct((N_TOKENS, DIM), jnp.float32),
        scratch_types=[
            pltpu.VMEM((b_per_w,), jnp.int32),
            pltpu.VMEM((b_per_w, DIM), jnp.float32),
            pltpu.SemaphoreType.DMA,
        ],
    )
    def gather(table_hbm, idx_hbm, out_hbm, idx_v, rows_v, sem):
        wid = lax.axis_index("s") * nc + lax.axis_index("c")
        base = wid * b_per_w
        pltpu.sync_copy(idx_hbm.at[pl.ds(base, b_per_w)], idx_v)
        pltpu.async_copy(table_hbm.at[idx_v], rows_v, sem).wait()
        pltpu.sync_copy(rows_v, out_hbm.at[pl.ds(base, b_per_w)])

    return gather


def kernel(x, codebook, W):
    b, n, d = x.shape
    xf = x.reshape(b * n, d)
    implicit, idx3, loss = _fused_call(xf, codebook, W)
    idx = idx3.reshape(N_TOKENS)
    q = _make_gather()(implicit, idx)
    return q.reshape(x.shape), idx.reshape(b, n), loss[0, 0]
